# baseline (speedup 1.0000x reference)
import functools

import jax
import jax.numpy as jnp
from jax import lax
from jax.experimental import pallas as pl
from jax.experimental.pallas import tpu as pltpu
from jax.experimental.pallas import tpu_sc as plsc

D = 18432
B = 4096
NCORES = 2
NSUB = 16
NW = NCORES * NSUB
P_SC = 3072
E_PER_W = P_SC // NW
BATCH = 32
SEQ = 128


def _sc_gather(sidx2d, spos2d, firstf, table):
    mesh = plsc.VectorSubcoreMesh(core_axis_name="c", subcore_axis_name="s")

    @functools.partial(
        pl.kernel,
        out_type=jax.ShapeDtypeStruct((P_SC, D), jnp.float32),
        mesh=mesh,
        scratch_types=[
            pltpu.VMEM((E_PER_W, 1), jnp.int32),
            pltpu.VMEM((E_PER_W, 1), jnp.int32),
            pltpu.VMEM((E_PER_W,), jnp.int32),
            pltpu.VMEM((2, 1, D), jnp.float32),
            pltpu.SemaphoreType.DMA,
            pltpu.SemaphoreType.DMA,
            pltpu.SemaphoreType.DMA,
        ],
        compiler_params=pltpu.CompilerParams(needs_layout_passes=False),
    )
    def k(sidx_hbm, spos_hbm, first_hbm, table_hbm, out_hbm,
          idx_v, pos_v, flag_v, buf, gsem, ssem0, ssem1):
        wid = lax.axis_index("s") * NCORES + lax.axis_index("c")
        base = wid * E_PER_W

        pltpu.sync_copy(sidx_hbm.at[pl.ds(base, E_PER_W)], idx_v)
        pltpu.sync_copy(spos_hbm.at[pl.ds(base, E_PER_W)], pos_v)
        pltpu.sync_copy(first_hbm.at[pl.ds(base, E_PER_W)], flag_v)

        lanes = lax.iota(jnp.int32, 16)
        ssems = (ssem0, ssem1)

        def write(s, e):
            return pltpu.make_async_copy(
                buf.at[s], out_hbm.at[pos_v.at[e]], ssems[s])

        def drain(s, cnt):
            lax.fori_loop(0, cnt, lambda i, c: (write(s, 0).wait(), c)[1], 0)

        def step(e, carry):
            slot, cnt0, cnt1 = carry
            group = flag_v[pl.ds((e // 16) * 16, 16)]
            is_new = jnp.logical_or(
                jnp.any(jnp.logical_and(group != 0, lanes == e % 16)),
                e == 0)
            nslot = jnp.where(is_new, 1 - slot, slot)

            for s in (0, 1):
                @pl.when(jnp.logical_and(is_new, nslot == s))
                def _():
                    drain(s, cnt0 if s == 0 else cnt1)
                    pltpu.async_copy(
                        table_hbm.at[idx_v.at[e]], buf.at[s], gsem).wait()

            for s in (0, 1):
                @pl.when(nslot == s)
                def _():
                    write(s, e).start()

            on0 = (nslot == 0).astype(jnp.int32)
            ncnt0 = jnp.where(jnp.logical_and(is_new, nslot == 0),
                              jnp.int32(0), cnt0) + on0
            ncnt1 = jnp.where(jnp.logical_and(is_new, nslot == 1),
                              jnp.int32(0), cnt1) + (1 - on0)
            return nslot, ncnt0, ncnt1

        _, cnt0, cnt1 = lax.fori_loop(
            0, E_PER_W, step,
            (jnp.int32(1), jnp.int32(0), jnp.int32(0)))
        drain(0, cnt0)
        drain(1, cnt1)

    return k(sidx2d, spos2d, firstf, table)


def kernel(prefix, embedding):
    flat = prefix.reshape(B)
    head = flat[:P_SC].astype(jnp.uint32)
    pos = lax.iota(jnp.uint32, P_SC)
    key = jnp.sort((head << jnp.uint32(12)) | pos)
    sidx = (key >> jnp.uint32(12)).astype(jnp.int32)
    spos = (key & jnp.uint32(4095)).astype(jnp.int32)
    firstf = jnp.concatenate(
        [jnp.ones((1,), jnp.int32),
         (sidx[1:] != sidx[:-1]).astype(jnp.int32)])
    out_sc = _sc_gather(sidx.reshape(P_SC, 1), spos.reshape(P_SC, 1),
                        firstf, embedding)
    out_tc = jnp.take(embedding, flat[P_SC:], axis=0)
    out = jnp.concatenate([out_sc, out_tc], axis=0)
    return out.reshape(BATCH, SEQ, D)


# speedup vs baseline: 2.5215x; 2.5215x over previous
import functools

import jax
import jax.numpy as jnp
from jax import lax
from jax.experimental import pallas as pl
from jax.experimental.pallas import tpu as pltpu
from jax.experimental.pallas import tpu_sc as plsc

D = 18432
B = 4096
NCORES = 2
NSUB = 16
NW = NCORES * NSUB
E_PER_W = B // NW
BATCH = 32
SEQ = 128


def _sc_gather(sidx2d, spos2d, firstf, table):
    mesh = plsc.VectorSubcoreMesh(core_axis_name="c", subcore_axis_name="s")

    @functools.partial(
        pl.kernel,
        out_type=jax.ShapeDtypeStruct((B, D), jnp.float32),
        mesh=mesh,
        scratch_types=[
            pltpu.VMEM((E_PER_W, 1), jnp.int32),
            pltpu.VMEM((E_PER_W, 1), jnp.int32),
            pltpu.VMEM((E_PER_W,), jnp.int32),
            pltpu.VMEM((2, 1, D), jnp.float32),
            pltpu.SemaphoreType.DMA,
            pltpu.SemaphoreType.DMA,
            pltpu.SemaphoreType.DMA,
        ],
        compiler_params=pltpu.CompilerParams(needs_layout_passes=False),
    )
    def k(sidx_hbm, spos_hbm, first_hbm, table_hbm, out_hbm,
          idx_v, pos_v, flag_v, buf, gsem, ssem0, ssem1):
        wid = lax.axis_index("s") * NCORES + lax.axis_index("c")
        base = wid * E_PER_W

        pltpu.sync_copy(sidx_hbm.at[pl.ds(base, E_PER_W)], idx_v)
        pltpu.sync_copy(spos_hbm.at[pl.ds(base, E_PER_W)], pos_v)
        pltpu.sync_copy(first_hbm.at[pl.ds(base, E_PER_W)], flag_v)

        lanes = lax.iota(jnp.int32, 16)
        ssems = (ssem0, ssem1)

        def write(s, e):
            return pltpu.make_async_copy(
                buf.at[s], out_hbm.at[pos_v.at[e]], ssems[s])

        def drain(s, cnt):
            lax.fori_loop(0, cnt, lambda i, c: (write(s, 0).wait(), c)[1], 0)

        def step(e, carry):
            slot, cnt0, cnt1 = carry
            group = flag_v[pl.ds((e // 16) * 16, 16)]
            is_new = jnp.logical_or(
                jnp.any(jnp.logical_and(group != 0, lanes == e % 16)),
                e == 0)
            nslot = jnp.where(is_new, 1 - slot, slot)

            for s in (0, 1):
                @pl.when(jnp.logical_and(is_new, nslot == s))
                def _():
                    drain(s, cnt0 if s == 0 else cnt1)
                    pltpu.async_copy(
                        table_hbm.at[idx_v.at[e]], buf.at[s], gsem).wait()

            for s in (0, 1):
                @pl.when(nslot == s)
                def _():
                    write(s, e).start()

            on0 = (nslot == 0).astype(jnp.int32)
            ncnt0 = jnp.where(jnp.logical_and(is_new, nslot == 0),
                              jnp.int32(0), cnt0) + on0
            ncnt1 = jnp.where(jnp.logical_and(is_new, nslot == 1),
                              jnp.int32(0), cnt1) + (1 - on0)
            return nslot, ncnt0, ncnt1

        _, cnt0, cnt1 = lax.fori_loop(
            0, E_PER_W, step,
            (jnp.int32(1), jnp.int32(0), jnp.int32(0)))
        drain(0, cnt0)
        drain(1, cnt1)

    return k(sidx2d, spos2d, firstf, table)


def kernel(prefix, embedding):
    flat = prefix.reshape(B).astype(jnp.uint32)
    pos = lax.iota(jnp.uint32, B)
    key = jnp.sort((flat << jnp.uint32(12)) | pos)
    sidx = (key >> jnp.uint32(12)).astype(jnp.int32)
    spos = (key & jnp.uint32(4095)).astype(jnp.int32)
    firstf = jnp.concatenate(
        [jnp.ones((1,), jnp.int32),
         (sidx[1:] != sidx[:-1]).astype(jnp.int32)])
    out = _sc_gather(sidx.reshape(B, 1), spos.reshape(B, 1), firstf,
                     embedding)
    return out.reshape(BATCH, SEQ, D)
